# Initial kernel scaffold; baseline (speedup 1.0000x reference)
import jax
import jax.numpy as jnp
from jax.experimental import pallas as pl


def kernel(x, edge_index, graph_indices, W1, b1, W2, b2, Wv, bv):
    raise NotImplementedError("write your pallas kernel here")


# speedup vs baseline: 5.8539x; 5.8539x over previous
import functools

import jax
import jax.numpy as jnp
from jax import lax
from jax.experimental import pallas as pl
from jax.experimental.pallas import tpu as pltpu
from jax.experimental.pallas import tpu_sc as plsc

N = 10000
E = 320000
D = 128
H = 128
B = 64

NP = 10240
NC = 2
NS = 16
NW = NC * NS
K = 128
EP = 327680
CHUNKS = EP // (NW * K)
ROWS_PER_TILE = NP // NS

_mesh = plsc.VectorSubcoreMesh(core_axis_name="c", subcore_axis_name="s",
                               num_cores=NC, num_subcores=NS)


IB = 40


@functools.partial(
    pl.kernel,
    out_type=jax.ShapeDtypeStruct((NC, NP, D), jnp.float32),
    mesh=_mesh,
    scratch_types=[
        pltpu.VMEM((IB, K), jnp.int32),
        pltpu.VMEM((IB, K), jnp.int32),
        pltpu.VMEM((K, D), jnp.float32),
        pltpu.VMEM((K, D), jnp.float32),
        pltpu.VMEM_SHARED((NP, D), jnp.float32),
        pltpu.SemaphoreType.DMA,
        pltpu.SemaphoreType.DMA,
    ],
)
def _seg_sum_rows(x_hbm, src_hbm, dst_hbm, zeros_hbm, out_hbm,
                  srcv, dstv, bufa, bufb, acc, sema, semb):
    c = lax.axis_index("c")
    s = lax.axis_index("s")
    w = s * NC + c
    pltpu.sync_copy(zeros_hbm.at[pl.ds(s * ROWS_PER_TILE, ROWS_PER_TILE)],
                    acc.at[pl.ds(s * ROWS_PER_TILE, ROWS_PER_TILE)])
    plsc.subcore_barrier()

    for blk in range(CHUNKS // IB):
        base = w * CHUNKS + blk * IB
        pltpu.sync_copy(src_hbm.at[pl.ds(base, IB)], srcv)
        pltpu.sync_copy(dst_hbm.at[pl.ds(base, IB)], dstv)

        pltpu.async_copy(x_hbm.at[srcv.at[0]], bufa, sema)

        def body(j, _):
            jj = 2 * j
            pltpu.async_copy(x_hbm.at[srcv.at[jj + 1]], bufb, semb)
            pltpu.make_async_copy(x_hbm.at[srcv.at[jj]], bufa, sema).wait()
            pltpu.sync_copy(bufa, acc.at[dstv.at[jj]], add=True)

            @pl.when(jj + 2 < IB)
            def _():
                pltpu.async_copy(x_hbm.at[srcv.at[jj + 2]], bufa, sema)

            pltpu.make_async_copy(x_hbm.at[srcv.at[jj + 1]], bufb, semb).wait()
            pltpu.sync_copy(bufb, acc.at[dstv.at[jj + 1]], add=True)
            return _

        lax.fori_loop(0, IB // 2, body, None)

    plsc.subcore_barrier()
    pltpu.sync_copy(acc.at[pl.ds(s * ROWS_PER_TILE, ROWS_PER_TILE)],
                    out_hbm.at[c, pl.ds(s * ROWS_PER_TILE, ROWS_PER_TILE)])


@functools.partial(
    pl.kernel,
    out_type=jax.ShapeDtypeStruct((NC, NP), jnp.float32),
    mesh=_mesh,
    scratch_types=[
        pltpu.VMEM((CHUNKS, K), jnp.int32),
        pltpu.VMEM((CHUNKS, K), jnp.int32),
        pltpu.VMEM((K,), jnp.float32),
        pltpu.VMEM((K,), jnp.float32),
        pltpu.VMEM_SHARED((NP,), jnp.float32),
        pltpu.SemaphoreType.DMA,
        pltpu.SemaphoreType.DMA,
    ],
)
def _seg_sum_scalar(s_hbm, src_hbm, dst_hbm, zeros_hbm, out_hbm,
                    srcv, dstv, bufa, bufb, acc, sema, semb):
    c = lax.axis_index("c")
    s = lax.axis_index("s")
    w = s * NC + c
    pltpu.sync_copy(zeros_hbm.at[pl.ds(s * ROWS_PER_TILE, ROWS_PER_TILE)],
                    acc.at[pl.ds(s * ROWS_PER_TILE, ROWS_PER_TILE)])
    pltpu.sync_copy(src_hbm.at[pl.ds(w * CHUNKS, CHUNKS)], srcv)
    pltpu.sync_copy(dst_hbm.at[pl.ds(w * CHUNKS, CHUNKS)], dstv)
    plsc.subcore_barrier()

    pltpu.async_copy(s_hbm.at[srcv.at[0]], bufa, sema)

    def body(j, _):
        jj = 2 * j

        @pl.when(jj + 1 < CHUNKS)
        def _():
            pltpu.async_copy(s_hbm.at[srcv.at[jj + 1]], bufb, semb)

        pltpu.make_async_copy(s_hbm.at[srcv.at[jj]], bufa, sema).wait()
        pltpu.sync_copy(bufa, acc.at[dstv.at[jj]], add=True)

        @pl.when(jj + 2 < CHUNKS)
        def _():
            pltpu.async_copy(s_hbm.at[srcv.at[jj + 2]], bufa, sema)

        @pl.when(jj + 1 < CHUNKS)
        def _():
            pltpu.make_async_copy(s_hbm.at[srcv.at[jj + 1]], bufb, semb).wait()
            pltpu.sync_copy(bufb, acc.at[dstv.at[jj + 1]], add=True)

        return _

    lax.fori_loop(0, (CHUNKS + 1) // 2, body, None)
    plsc.subcore_barrier()
    pltpu.sync_copy(acc.at[pl.ds(s * ROWS_PER_TILE, ROWS_PER_TILE)],
                    out_hbm.at[c, pl.ds(s * ROWS_PER_TILE, ROWS_PER_TILE)])


_RB = 1024


def _embed_body(x_ref, p_ref, w1_ref, b1_ref, w2_ref, g_ref, s_out, gp_out):
    i = pl.program_id(0)
    xa = x_ref[...] + p_ref[0] + p_ref[1]
    e = jnp.maximum(
        jnp.dot(xa, w1_ref[...], preferred_element_type=jnp.float32)
        + b1_ref[...], 0.0)
    s_out[...] = jnp.dot(e, w2_ref[...], preferred_element_type=jnp.float32)
    oh = (g_ref[...] == lax.broadcasted_iota(jnp.int32, (_RB, B), 1)
          ).astype(jnp.float32)
    gp = lax.dot_general(oh, e, (((0,), (0,)), ((), ())),
                         preferred_element_type=jnp.float32)

    @pl.when(i == 0)
    def _():
        gp_out[...] = gp

    @pl.when(i > 0)
    def _():
        gp_out[...] += gp


def _embed_stage():
    return pl.pallas_call(
        _embed_body,
        grid=(NP // _RB,),
        in_specs=[
            pl.BlockSpec((_RB, D), lambda i: (i, 0)),
            pl.BlockSpec((NC, _RB, D), lambda i: (0, i, 0)),
            pl.BlockSpec((D, H), lambda i: (0, 0)),
            pl.BlockSpec((1, H), lambda i: (0, 0)),
            pl.BlockSpec((H, 1), lambda i: (0, 0)),
            pl.BlockSpec((_RB, 1), lambda i: (i, 0)),
        ],
        out_specs=[
            pl.BlockSpec((_RB, 1), lambda i: (i, 0)),
            pl.BlockSpec((B, H), lambda i: (0, 0)),
        ],
        out_shape=[
            jax.ShapeDtypeStruct((NP, 1), jnp.float32),
            jax.ShapeDtypeStruct((B, H), jnp.float32),
        ],
    )


def _merge_body(s_ref, t_ref, g_ref, gp_ref, wv_ref, bv_ref, b2_ref, out_ref):
    adv = 2.0 * jnp.tanh(s_ref[...] + t_ref[0] + t_ref[1] + b2_ref[0, 0])
    oh = (g_ref[...] == lax.broadcasted_iota(jnp.int32, (NP, B), 1)
          ).astype(jnp.float32)
    adv_sum = lax.dot_general(oh, adv, (((0,), (0,)), ((), ())),
                              preferred_element_type=jnp.float32)
    counts = lax.dot_general(oh, jnp.ones_like(adv), (((0,), (0,)), ((), ())),
                             preferred_element_type=jnp.float32)
    value = jnp.tanh(
        jnp.dot(gp_ref[...], wv_ref[...], preferred_element_type=jnp.float32)
        + bv_ref[0, 0])
    vm = value - adv_sum / jnp.maximum(counts, 1.0)
    per_node = jnp.dot(oh, vm, preferred_element_type=jnp.float32)
    out_ref[...] = jnp.tanh(per_node + adv)


def kernel(x, edge_index, graph_indices, W1, b1, W2, b2, Wv, bv):
    f32 = jnp.float32
    pad_e = EP - E
    src2 = jnp.concatenate(
        [edge_index[0], jnp.zeros((pad_e,), jnp.int32)]).reshape(EP // K, K)
    dst2 = jnp.concatenate(
        [edge_index[1], jnp.full((pad_e,), NP - 1, jnp.int32)]
    ).reshape(EP // K, K)
    x_pad = jnp.pad(x, ((0, NP - N), (0, 0)))
    g_pad = jnp.pad(graph_indices, (0, NP - N), constant_values=B)
    zeros2d = jnp.zeros((NP, D), f32)
    zeros1d = jnp.zeros((NP,), f32)

    p = _seg_sum_rows(x_pad, src2, dst2, zeros2d)

    s_col, gp = _embed_stage()(x_pad, p, W1, b1[None, :], W2, g_pad[:, None])

    t = _seg_sum_scalar(s_col[:, 0], src2, dst2, zeros1d)

    out = pl.pallas_call(
        _merge_body,
        out_shape=jax.ShapeDtypeStruct((NP, 1), jnp.float32),
    )(s_col, t[:, :, None], g_pad[:, None], gp, Wv,
      bv[None, :], b2[None, :])
    return out[:N, 0]
